# baseline (speedup 1.0000x reference)
import functools

import jax
import jax.numpy as jnp
from jax import lax
from jax.experimental import pallas as pl
from jax.experimental.pallas import tpu as pltpu
from jax.experimental.pallas import tpu_sc as plsc

N = 10000
E = 320000
D = 128
C = 16
ALPHA = 0.1
REG = 1e-05

NW = 32
NP = 320
N_PAD = NW * NP
CH = 128
NPT = N_PAD // 16
TRASH = 128
E_PAD = 327680
EB_A = 20
NB0 = 16
NB1 = 4
R0, P0 = 16000, 384
R1, P1 = 4000, 96
E0 = 16 * R0


def _rsqrt16(v):
    i = lax.bitcast_convert_type(v, jnp.int32)
    i = jnp.int32(0x5F3759DF) - (i >> 1)
    y = lax.bitcast_convert_type(i, jnp.float32)
    for _ in range(3):
        y = y * (1.5 - 0.5 * v * y * y)
    return y


def _deg_dinv_s0_body(col3d_hbm, x_hbm, dinv_hbm, s0_hbm,
                      deg_sh, zbuf, ones_v, ci_all, degv, xv, sem_i, sem_s):
    cid = lax.axis_index("c")
    sid = lax.axis_index("s")
    wid = cid * 16 + sid

    for g in range(NPT // 16):
        zbuf[pl.ds(16 * g, 16)] = jnp.zeros((16,), jnp.float32)
    pltpu.sync_copy(zbuf, deg_sh.at[pl.ds(sid * NPT, NPT)])
    pltpu.sync_copy(zbuf.at[pl.ds(0, 8)],
                    deg_sh.at[pl.ds(N_PAD + sid * 8, 8)])
    for g in range(CH // 16):
        ones_v[pl.ds(16 * g, 16)] = jnp.full((16,), 1.0, jnp.float32)
    plsc.subcore_barrier()

    pltpu.async_copy(col3d_hbm.at[pl.ds(sid * EB_A, EB_A)], ci_all,
                     sem_i).wait()
    descs = []
    for b in range(EB_A):
        for t in range(8):
            descs.append(pltpu.async_copy(
                ones_v, deg_sh.at[ci_all.at[b, t]], sem_s, add=True))
    for dsc in descs:
        dsc.wait()
    plsc.subcore_barrier()

    base = wid * NP
    pltpu.sync_copy(deg_sh.at[pl.ds(base, NP)], degv)
    for g in range(NP // 16):
        ids = base + 16 * g + lax.iota(jnp.int32, 16)
        real = ids < N
        dv = degv[pl.ds(16 * g, 16)] + jnp.where(real, 1.0, 0.0)
        degv[pl.ds(16 * g, 16)] = jnp.where(real, _rsqrt16(dv), 0.0)
    pltpu.sync_copy(degv, dinv_hbm.at[pl.ds(base, NP)])

    pltpu.sync_copy(x_hbm.at[pl.ds(base, NP)], xv)

    def scale_group(g, _):
        dvec = degv[pl.ds(16 * g, 16)]
        for l in range(16):
            dv = dvec[l]
            r = 16 * g + l
            for j in range(D // 16):
                xv[r, pl.ds(16 * j, 16)] = xv[r, pl.ds(16 * j, 16)] * dv
        return _

    lax.fori_loop(0, NP // 16, scale_group, None)
    pltpu.sync_copy(xv, s0_hbm.at[pl.ds(base, NP)])


def _deg_dinv_s0(col3d, x_pad):
    mesh = plsc.VectorSubcoreMesh(core_axis_name="c", subcore_axis_name="s")
    return pl.kernel(
        _deg_dinv_s0_body,
        out_type=(
            jax.ShapeDtypeStruct((N_PAD,), jnp.float32),
            jax.ShapeDtypeStruct((N_PAD, D), jnp.float32),
        ),
        mesh=mesh,
        scratch_types=[
            pltpu.VMEM_SHARED((N_PAD + TRASH,), jnp.float32),
            pltpu.VMEM((NPT,), jnp.float32),
            pltpu.VMEM((CH,), jnp.float32),
            pltpu.VMEM((EB_A, 8, CH), jnp.int32),
            pltpu.VMEM((NP,), jnp.float32),
            pltpu.VMEM((NP, D), jnp.float32),
            pltpu.SemaphoreType.DMA,
            pltpu.SemaphoreType.DMA,
        ],
    )(col3d, x_pad)


def _hop_body(s_hbm, row3d_hbm, col3d_hbm, tp_hbm,
              acc_sh, ri0, ci0, ri1, ci1, rows0, rows1,
              sem_i0, sem_i1, sem_g0, sem_g1):
    cid = lax.axis_index("c")
    sid = lax.axis_index("s")

    def zrow(r, _):
        for g in range(D // 16):
            rows0[r, pl.ds(16 * g, 16)] = jnp.zeros((16,), jnp.float32)
        return _

    lax.fori_loop(0, CH, zrow, None)
    for b in range(NPT // CH):
        pltpu.sync_copy(rows0, acc_sh.at[pl.ds(sid * NPT + b * CH, CH)])
    pltpu.sync_copy(rows0.at[pl.ds(0, 8)],
                    acc_sh.at[pl.ds(N_PAD + sid * 8, 8)])

    ibufs = ((ri0, ci0), (ri1, ci1))
    isems = (sem_i0, sem_i1)
    gbufs = (rows0, rows1)
    gsems = (sem_g0, sem_g1)

    def run_pipe(tb, nblk):
        def fire_idx(blk, par):
            dr = pltpu.async_copy(row3d_hbm.at[tb + blk], ibufs[par][0],
                                  isems[par])
            dc = pltpu.async_copy(col3d_hbm.at[tb + blk], ibufs[par][1],
                                  isems[par])
            return (dr, dc)

        pend_i = [fire_idx(0, 0), None]
        for dsc in pend_i[0]:
            dsc.wait()
        if nblk > 1:
            pend_i[1] = fire_idx(1, 1)
        n_ch = nblk * 8
        pend_g = pltpu.async_copy(s_hbm.at[ri0.at[0]], rows0, sem_g0)
        for k in range(n_ch):
            b, t = divmod(k, 8)
            par = b % 2
            cur_buf = gbufs[k % 2]
            cur_dsc = pend_g
            if k + 1 < n_ch:
                nb, nt = divmod(k + 1, 8)
                if nt == 0:
                    for dsc in pend_i[nb % 2]:
                        dsc.wait()
                pend_g = pltpu.async_copy(s_hbm.at[ibufs[nb % 2][0].at[nt]],
                                          gbufs[(k + 1) % 2],
                                          gsems[(k + 1) % 2])
            cur_dsc.wait()
            pltpu.sync_copy(cur_buf, acc_sh.at[ibufs[par][1].at[t]], add=True)
            if t == 7 and b + 2 < nblk:
                pend_i[par] = fire_idx(b + 2, par)

    @pl.when(cid == 0)
    def _():
        run_pipe(sid * NB0, NB0)

    @pl.when(cid == 1)
    def _():
        run_pipe(16 * NB0 + sid * NB1, NB1)

    plsc.subcore_barrier()

    pltpu.sync_copy(acc_sh.at[pl.ds(sid * NPT, NPT)],
                    tp_hbm.at[cid, pl.ds(sid * NPT, NPT)])


def _hop(s, row3d, col3d):
    mesh = plsc.VectorSubcoreMesh(core_axis_name="c", subcore_axis_name="s")
    return pl.kernel(
        _hop_body,
        out_type=jax.ShapeDtypeStruct((2, N_PAD, D), jnp.float32),
        mesh=mesh,
        scratch_types=[
            pltpu.VMEM_SHARED((N_PAD + TRASH, D), jnp.float32),
            pltpu.VMEM((8, CH), jnp.int32),
            pltpu.VMEM((8, CH), jnp.int32),
            pltpu.VMEM((8, CH), jnp.int32),
            pltpu.VMEM((8, CH), jnp.int32),
            pltpu.VMEM((CH, D), jnp.float32),
            pltpu.VMEM((CH, D), jnp.float32),
            pltpu.SemaphoreType.DMA,
            pltpu.SemaphoreType.DMA,
            pltpu.SemaphoreType.DMA,
            pltpu.SemaphoreType.DMA,
        ],
    )(s, row3d, col3d)


BN = 2048
BN2 = 1000


def _combine1_body(tp0, tp1, s, x, dinv, s_next):
    t = tp0[...] + tp1[...] + s[...]
    dv = dinv[...]
    out = (1.0 - ALPHA) * (dv * t) + ALPHA * x[...]
    s_next[...] = dv * out


def _combine1(tp, s, x_pad, dinv_col):
    grid = (N_PAD // BN,)
    return pl.pallas_call(
        _combine1_body,
        grid=grid,
        in_specs=[
            pl.BlockSpec((BN, D), lambda i: (i, 0)),
            pl.BlockSpec((BN, D), lambda i: (i, 0)),
            pl.BlockSpec((BN, D), lambda i: (i, 0)),
            pl.BlockSpec((BN, D), lambda i: (i, 0)),
            pl.BlockSpec((BN, 1), lambda i: (i, 0)),
        ],
        out_specs=pl.BlockSpec((BN, D), lambda i: (i, 0)),
        out_shape=jax.ShapeDtypeStruct((N_PAD, D), jnp.float32),
    )(tp[0], tp[1], s, x_pad, dinv_col)


def _combine2_body(tp0, tp1, s, x, dinv, mask, yb, xg_out, g_out, r_out,
                   acc_g, acc_r):
    i = pl.program_id(0)

    @pl.when(i == 0)
    def _():
        acc_g[...] = jnp.zeros_like(acc_g)
        acc_r[...] = jnp.zeros_like(acc_r)

    t = tp0[...] + tp1[...] + s[...]
    dv = dinv[...]
    xg = (1.0 - ALPHA) * (dv * t) + ALPHA * x[...]
    xg_out[...] = xg
    xm = xg * mask[...]
    acc_g[...] += lax.dot_general(xm, xg, (((0,), (0,)), ((), ())),
                                  preferred_element_type=jnp.float32)
    acc_r[...] += lax.dot_general(xm, yb[...], (((0,), (0,)), ((), ())),
                                  preferred_element_type=jnp.float32)

    @pl.when(i == N_PAD // BN - 1)
    def _():
        rows = lax.broadcasted_iota(jnp.int32, (D, D), 0)
        cols = lax.broadcasted_iota(jnp.int32, (D, D), 1)
        eye = jnp.where(rows == cols, jnp.float32(REG), jnp.float32(0.0))
        g_out[...] = acc_g[...] + eye
        r_out[...] = acc_r[...]


def _combine2(tp, s, x_pad, dinv_col, mask_col, y_pad):
    grid = (N_PAD // BN,)
    return pl.pallas_call(
        _combine2_body,
        grid=grid,
        in_specs=[
            pl.BlockSpec((BN, D), lambda i: (i, 0)),
            pl.BlockSpec((BN, D), lambda i: (i, 0)),
            pl.BlockSpec((BN, D), lambda i: (i, 0)),
            pl.BlockSpec((BN, D), lambda i: (i, 0)),
            pl.BlockSpec((BN, 1), lambda i: (i, 0)),
            pl.BlockSpec((BN, 1), lambda i: (i, 0)),
            pl.BlockSpec((BN, C), lambda i: (i, 0)),
        ],
        out_specs=[
            pl.BlockSpec((BN, D), lambda i: (i, 0)),
            pl.BlockSpec((D, D), lambda i: (0, 0)),
            pl.BlockSpec((D, C), lambda i: (0, 0)),
        ],
        out_shape=[
            jax.ShapeDtypeStruct((N_PAD, D), jnp.float32),
            jax.ShapeDtypeStruct((D, D), jnp.float32),
            jax.ShapeDtypeStruct((D, C), jnp.float32),
        ],
        scratch_shapes=[
            pltpu.VMEM((D, D), jnp.float32),
            pltpu.VMEM((D, C), jnp.float32),
        ],
    )(tp[0], tp[1], s, x_pad, dinv_col, mask_col, y_pad)


def _solve_predict_body(g_ref, r_ref, xg, yp, sol):
    i = pl.program_id(0)

    @pl.when(i == 0)
    def _():
        a = g_ref[...]
        aabs = jnp.abs(a)
        n1 = jnp.max(jnp.sum(aabs, axis=0))
        ninf = jnp.max(jnp.sum(aabs, axis=1))
        rows = lax.broadcasted_iota(jnp.int32, (D, D), 0)
        cols = lax.broadcasted_iota(jnp.int32, (D, D), 1)
        two_i = jnp.where(rows == cols, jnp.float32(2.0), jnp.float32(0.0))
        x0 = a * (1.0 / (n1 * ninf))

        xinv = x0
        for _ in range(24):
            ax = lax.dot_general(a, xinv, (((1,), (0,)), ((), ())),
                                 preferred_element_type=jnp.float32, precision=lax.Precision.HIGHEST)
            xinv = lax.dot_general(xinv, two_i - ax, (((1,), (0,)), ((), ())),
                                   preferred_element_type=jnp.float32, precision=lax.Precision.HIGHEST)
        sol[...] = lax.dot_general(xinv, r_ref[...], (((1,), (0,)), ((), ())),
                                   preferred_element_type=jnp.float32)

    yp[...] = lax.dot_general(xg[...], sol[...], (((1,), (0,)), ((), ())),
                              preferred_element_type=jnp.float32)


def _solve_predict(g_mat, r_mat, xg):
    grid = (N // BN2,)
    return pl.pallas_call(
        _solve_predict_body,
        grid=grid,
        in_specs=[
            pl.BlockSpec((D, D), lambda i: (0, 0)),
            pl.BlockSpec((D, C), lambda i: (0, 0)),
            pl.BlockSpec((BN2, D), lambda i: (i, 0)),
        ],
        out_specs=pl.BlockSpec((BN2, C), lambda i: (i, 0)),
        out_shape=jax.ShapeDtypeStruct((N, C), jnp.float32),
        scratch_shapes=[pltpu.VMEM((D, C), jnp.float32)],
    )(g_mat, r_mat, xg)


def kernel(x, edge_index, y_one_hot, train_mask):
    w0 = jnp.arange(16, dtype=jnp.int32)[:, None]
    w1 = w0 + 16
    i0 = jnp.arange(P0, dtype=jnp.int32)[None, :]
    i1 = jnp.arange(P1, dtype=jnp.int32)[None, :]

    def build(idx, pad0, pad1):
        a0 = jnp.concatenate([idx[:E0].reshape(16, R0), pad0], axis=1)
        a1 = jnp.concatenate([idx[E0:].reshape(16, R1), pad1], axis=1)
        return jnp.concatenate(
            [a0.reshape(-1, 8, CH), a1.reshape(-1, 8, CH)], axis=0)

    row3d = build(edge_index[0],
                  (w0 * 7919 + i0 * 41) % N, (w1 * 7919 + i1 * 41) % N)
    col3d = build(edge_index[1],
                  N_PAD + ((w0 * 8 + i0) % TRASH),
                  N_PAD + ((w1 * 8 + i1) % TRASH))
    x_pad = jnp.pad(x, ((0, N_PAD - N), (0, 0)))
    y_pad = jnp.pad(y_one_hot, ((0, N_PAD - N), (0, 0)))
    mask_col = jnp.pad(train_mask.astype(jnp.float32), (0, N_PAD - N))[:, None]

    dinv, s0 = _deg_dinv_s0(col3d, x_pad)
    dinv_col = dinv[:, None]

    tp1 = _hop(s0, row3d, col3d)
    s1 = _combine1(tp1, s0, x_pad, dinv_col)
    tp2 = _hop(s1, row3d, col3d)
    xg, g_mat, r_mat = _combine2(tp2, s1, x_pad, dinv_col, mask_col, y_pad)
    return _solve_predict(g_mat, r_mat, xg)


# speedup vs baseline: 1.2181x; 1.2181x over previous
import functools

import jax
import jax.numpy as jnp
from jax import lax
from jax.experimental import pallas as pl
from jax.experimental.pallas import tpu as pltpu
from jax.experimental.pallas import tpu_sc as plsc

N = 10000
E = 320000
D = 128
C = 16
ALPHA = 0.1
REG = 1e-05

NW = 32
NP = 320
N_PAD = NW * NP
CH = 128
NPT = N_PAD // 16
TRASH = 128
E_PAD = 327680
EB_A = 20
NB0 = 12
NB1 = 8
R0, P0 = 12000, 288
R1, P1 = 8000, 192
E0 = 16 * R0


def _rsqrt16(v):
    i = lax.bitcast_convert_type(v, jnp.int32)
    i = jnp.int32(0x5F3759DF) - (i >> 1)
    y = lax.bitcast_convert_type(i, jnp.float32)
    for _ in range(3):
        y = y * (1.5 - 0.5 * v * y * y)
    return y


def _deg_dinv_s0_body(col3d_hbm, x_hbm, dinv_hbm, s0_hbm,
                      deg_sh, zbuf, ones_v, ci_all, degv, xv, sem_i, sem_s):
    cid = lax.axis_index("c")
    sid = lax.axis_index("s")
    wid = cid * 16 + sid

    for g in range(NPT // 16):
        zbuf[pl.ds(16 * g, 16)] = jnp.zeros((16,), jnp.float32)
    pltpu.sync_copy(zbuf, deg_sh.at[pl.ds(sid * NPT, NPT)])
    pltpu.sync_copy(zbuf.at[pl.ds(0, 8)],
                    deg_sh.at[pl.ds(N_PAD + sid * 8, 8)])
    for g in range(CH // 16):
        ones_v[pl.ds(16 * g, 16)] = jnp.full((16,), 1.0, jnp.float32)
    plsc.subcore_barrier()

    pltpu.async_copy(col3d_hbm.at[pl.ds(sid * EB_A, EB_A)], ci_all,
                     sem_i).wait()
    descs = []
    for b in range(EB_A):
        for t in range(8):
            descs.append(pltpu.async_copy(
                ones_v, deg_sh.at[ci_all.at[b, t]], sem_s, add=True))
    for dsc in descs:
        dsc.wait()
    plsc.subcore_barrier()

    base = wid * NP
    pltpu.sync_copy(deg_sh.at[pl.ds(base, NP)], degv)
    for g in range(NP // 16):
        ids = base + 16 * g + lax.iota(jnp.int32, 16)
        real = ids < N
        dv = degv[pl.ds(16 * g, 16)] + jnp.where(real, 1.0, 0.0)
        degv[pl.ds(16 * g, 16)] = jnp.where(real, _rsqrt16(dv), 0.0)
    pltpu.sync_copy(degv, dinv_hbm.at[pl.ds(base, NP)])

    pltpu.sync_copy(x_hbm.at[pl.ds(base, NP)], xv)

    def scale_group(g, _):
        dvec = degv[pl.ds(16 * g, 16)]
        for l in range(16):
            dv = dvec[l]
            r = 16 * g + l
            for j in range(D // 16):
                xv[r, pl.ds(16 * j, 16)] = xv[r, pl.ds(16 * j, 16)] * dv
        return _

    lax.fori_loop(0, NP // 16, scale_group, None)
    pltpu.sync_copy(xv, s0_hbm.at[pl.ds(base, NP)])


def _deg_dinv_s0(col3d, x_pad):
    mesh = plsc.VectorSubcoreMesh(core_axis_name="c", subcore_axis_name="s")
    return pl.kernel(
        _deg_dinv_s0_body,
        out_type=(
            jax.ShapeDtypeStruct((N_PAD,), jnp.float32),
            jax.ShapeDtypeStruct((N_PAD, D), jnp.float32),
        ),
        mesh=mesh,
        scratch_types=[
            pltpu.VMEM_SHARED((N_PAD + TRASH,), jnp.float32),
            pltpu.VMEM((NPT,), jnp.float32),
            pltpu.VMEM((CH,), jnp.float32),
            pltpu.VMEM((EB_A, 8, CH), jnp.int32),
            pltpu.VMEM((NP,), jnp.float32),
            pltpu.VMEM((NP, D), jnp.float32),
            pltpu.SemaphoreType.DMA,
            pltpu.SemaphoreType.DMA,
        ],
    )(col3d, x_pad)


def _hop_body(s_hbm, row3d_hbm, col3d_hbm, tp_hbm,
              acc_sh, ri0, ci0, ri1, ci1, rows0, rows1,
              sem_i0, sem_i1, sem_g0, sem_g1):
    cid = lax.axis_index("c")
    sid = lax.axis_index("s")

    def zrow(r, _):
        for g in range(D // 16):
            rows0[r, pl.ds(16 * g, 16)] = jnp.zeros((16,), jnp.float32)
        return _

    lax.fori_loop(0, CH, zrow, None)
    for b in range(NPT // CH):
        pltpu.sync_copy(rows0, acc_sh.at[pl.ds(sid * NPT + b * CH, CH)])
    pltpu.sync_copy(rows0.at[pl.ds(0, 8)],
                    acc_sh.at[pl.ds(N_PAD + sid * 8, 8)])

    ibufs = ((ri0, ci0), (ri1, ci1))
    isems = (sem_i0, sem_i1)
    gbufs = (rows0, rows1)
    gsems = (sem_g0, sem_g1)

    def run_pipe(tb, nblk):
        def fire_idx(blk, par):
            dr = pltpu.async_copy(row3d_hbm.at[tb + blk], ibufs[par][0],
                                  isems[par])
            dc = pltpu.async_copy(col3d_hbm.at[tb + blk], ibufs[par][1],
                                  isems[par])
            return (dr, dc)

        pend_i = [fire_idx(0, 0), None]
        for dsc in pend_i[0]:
            dsc.wait()
        if nblk > 1:
            pend_i[1] = fire_idx(1, 1)
        n_ch = nblk * 8
        pend_g = pltpu.async_copy(s_hbm.at[ri0.at[0]], rows0, sem_g0)
        for k in range(n_ch):
            b, t = divmod(k, 8)
            par = b % 2
            cur_buf = gbufs[k % 2]
            cur_dsc = pend_g
            if k + 1 < n_ch:
                nb, nt = divmod(k + 1, 8)
                if nt == 0:
                    for dsc in pend_i[nb % 2]:
                        dsc.wait()
                pend_g = pltpu.async_copy(s_hbm.at[ibufs[nb % 2][0].at[nt]],
                                          gbufs[(k + 1) % 2],
                                          gsems[(k + 1) % 2])
            cur_dsc.wait()
            pltpu.sync_copy(cur_buf, acc_sh.at[ibufs[par][1].at[t]], add=True)
            if t == 7 and b + 2 < nblk:
                pend_i[par] = fire_idx(b + 2, par)

    @pl.when(cid == 0)
    def _():
        run_pipe(sid * NB0, NB0)

    @pl.when(cid == 1)
    def _():
        run_pipe(16 * NB0 + sid * NB1, NB1)

    plsc.subcore_barrier()

    pltpu.sync_copy(acc_sh.at[pl.ds(sid * NPT, NPT)],
                    tp_hbm.at[cid, pl.ds(sid * NPT, NPT)])


def _hop(s, row3d, col3d):
    mesh = plsc.VectorSubcoreMesh(core_axis_name="c", subcore_axis_name="s")
    return pl.kernel(
        _hop_body,
        out_type=jax.ShapeDtypeStruct((2, N_PAD, D), jnp.float32),
        mesh=mesh,
        scratch_types=[
            pltpu.VMEM_SHARED((N_PAD + TRASH, D), jnp.float32),
            pltpu.VMEM((8, CH), jnp.int32),
            pltpu.VMEM((8, CH), jnp.int32),
            pltpu.VMEM((8, CH), jnp.int32),
            pltpu.VMEM((8, CH), jnp.int32),
            pltpu.VMEM((CH, D), jnp.float32),
            pltpu.VMEM((CH, D), jnp.float32),
            pltpu.SemaphoreType.DMA,
            pltpu.SemaphoreType.DMA,
            pltpu.SemaphoreType.DMA,
            pltpu.SemaphoreType.DMA,
        ],
    )(s, row3d, col3d)


BN = 2048
BN2 = 1000


def _combine1_body(tp0, tp1, s, x, dinv, s_next):
    t = tp0[...] + tp1[...] + s[...]
    dv = dinv[...]
    out = (1.0 - ALPHA) * (dv * t) + ALPHA * x[...]
    s_next[...] = dv * out


def _combine1(tp, s, x_pad, dinv_col):
    grid = (N_PAD // BN,)
    return pl.pallas_call(
        _combine1_body,
        grid=grid,
        in_specs=[
            pl.BlockSpec((BN, D), lambda i: (i, 0)),
            pl.BlockSpec((BN, D), lambda i: (i, 0)),
            pl.BlockSpec((BN, D), lambda i: (i, 0)),
            pl.BlockSpec((BN, D), lambda i: (i, 0)),
            pl.BlockSpec((BN, 1), lambda i: (i, 0)),
        ],
        out_specs=pl.BlockSpec((BN, D), lambda i: (i, 0)),
        out_shape=jax.ShapeDtypeStruct((N_PAD, D), jnp.float32),
    )(tp[0], tp[1], s, x_pad, dinv_col)


def _combine2_body(tp0, tp1, s, x, dinv, mask, yb, xg_out, g_out, r_out,
                   acc_g, acc_r):
    i = pl.program_id(0)

    @pl.when(i == 0)
    def _():
        acc_g[...] = jnp.zeros_like(acc_g)
        acc_r[...] = jnp.zeros_like(acc_r)

    t = tp0[...] + tp1[...] + s[...]
    dv = dinv[...]
    xg = (1.0 - ALPHA) * (dv * t) + ALPHA * x[...]
    xg_out[...] = xg
    xm = xg * mask[...]
    acc_g[...] += lax.dot_general(xm, xg, (((0,), (0,)), ((), ())),
                                  preferred_element_type=jnp.float32)
    acc_r[...] += lax.dot_general(xm, yb[...], (((0,), (0,)), ((), ())),
                                  preferred_element_type=jnp.float32)

    @pl.when(i == N_PAD // BN - 1)
    def _():
        rows = lax.broadcasted_iota(jnp.int32, (D, D), 0)
        cols = lax.broadcasted_iota(jnp.int32, (D, D), 1)
        eye = jnp.where(rows == cols, jnp.float32(REG), jnp.float32(0.0))
        g_out[...] = acc_g[...] + eye
        r_out[...] = acc_r[...]


def _combine2(tp, s, x_pad, dinv_col, mask_col, y_pad):
    grid = (N_PAD // BN,)
    return pl.pallas_call(
        _combine2_body,
        grid=grid,
        in_specs=[
            pl.BlockSpec((BN, D), lambda i: (i, 0)),
            pl.BlockSpec((BN, D), lambda i: (i, 0)),
            pl.BlockSpec((BN, D), lambda i: (i, 0)),
            pl.BlockSpec((BN, D), lambda i: (i, 0)),
            pl.BlockSpec((BN, 1), lambda i: (i, 0)),
            pl.BlockSpec((BN, 1), lambda i: (i, 0)),
            pl.BlockSpec((BN, C), lambda i: (i, 0)),
        ],
        out_specs=[
            pl.BlockSpec((BN, D), lambda i: (i, 0)),
            pl.BlockSpec((D, D), lambda i: (0, 0)),
            pl.BlockSpec((D, C), lambda i: (0, 0)),
        ],
        out_shape=[
            jax.ShapeDtypeStruct((N_PAD, D), jnp.float32),
            jax.ShapeDtypeStruct((D, D), jnp.float32),
            jax.ShapeDtypeStruct((D, C), jnp.float32),
        ],
        scratch_shapes=[
            pltpu.VMEM((D, D), jnp.float32),
            pltpu.VMEM((D, C), jnp.float32),
        ],
    )(tp[0], tp[1], s, x_pad, dinv_col, mask_col, y_pad)


def _solve_predict_body(g_ref, r_ref, xg, yp, sol):
    i = pl.program_id(0)

    @pl.when(i == 0)
    def _():
        a = g_ref[...]
        aabs = jnp.abs(a)
        n1 = jnp.max(jnp.sum(aabs, axis=0))
        ninf = jnp.max(jnp.sum(aabs, axis=1))
        rows = lax.broadcasted_iota(jnp.int32, (D, D), 0)
        cols = lax.broadcasted_iota(jnp.int32, (D, D), 1)
        two_i = jnp.where(rows == cols, jnp.float32(2.0), jnp.float32(0.0))
        x0 = a * (1.0 / (n1 * ninf))

        xinv = x0
        for _ in range(24):
            ax = lax.dot_general(a, xinv, (((1,), (0,)), ((), ())),
                                 preferred_element_type=jnp.float32, precision=lax.Precision.HIGHEST)
            xinv = lax.dot_general(xinv, two_i - ax, (((1,), (0,)), ((), ())),
                                   preferred_element_type=jnp.float32, precision=lax.Precision.HIGHEST)
        sol[...] = lax.dot_general(xinv, r_ref[...], (((1,), (0,)), ((), ())),
                                   preferred_element_type=jnp.float32)

    yp[...] = lax.dot_general(xg[...], sol[...], (((1,), (0,)), ((), ())),
                              preferred_element_type=jnp.float32)


def _solve_predict(g_mat, r_mat, xg):
    grid = (N // BN2,)
    return pl.pallas_call(
        _solve_predict_body,
        grid=grid,
        in_specs=[
            pl.BlockSpec((D, D), lambda i: (0, 0)),
            pl.BlockSpec((D, C), lambda i: (0, 0)),
            pl.BlockSpec((BN2, D), lambda i: (i, 0)),
        ],
        out_specs=pl.BlockSpec((BN2, C), lambda i: (i, 0)),
        out_shape=jax.ShapeDtypeStruct((N, C), jnp.float32),
        scratch_shapes=[pltpu.VMEM((D, C), jnp.float32)],
    )(g_mat, r_mat, xg)


def kernel(x, edge_index, y_one_hot, train_mask):
    w0 = jnp.arange(16, dtype=jnp.int32)[:, None]
    w1 = w0 + 16
    i0 = jnp.arange(P0, dtype=jnp.int32)[None, :]
    i1 = jnp.arange(P1, dtype=jnp.int32)[None, :]

    def build(idx, pad0, pad1):
        a0 = jnp.concatenate([idx[:E0].reshape(16, R0), pad0], axis=1)
        a1 = jnp.concatenate([idx[E0:].reshape(16, R1), pad1], axis=1)
        return jnp.concatenate(
            [a0.reshape(-1, 8, CH), a1.reshape(-1, 8, CH)], axis=0)

    row3d = build(edge_index[0],
                  (w0 * 7919 + i0 * 41) % N, (w1 * 7919 + i1 * 41) % N)
    col3d = build(edge_index[1],
                  N_PAD + ((w0 * 8 + i0) % TRASH),
                  N_PAD + ((w1 * 8 + i1) % TRASH))
    x_pad = jnp.pad(x, ((0, N_PAD - N), (0, 0)))
    y_pad = jnp.pad(y_one_hot, ((0, N_PAD - N), (0, 0)))
    mask_col = jnp.pad(train_mask.astype(jnp.float32), (0, N_PAD - N))[:, None]

    dinv, s0 = _deg_dinv_s0(col3d, x_pad)
    dinv_col = dinv[:, None]

    tp1 = _hop(s0, row3d, col3d)
    s1 = _combine1(tp1, s0, x_pad, dinv_col)
    tp2 = _hop(s1, row3d, col3d)
    xg, g_mat, r_mat = _combine2(tp2, s1, x_pad, dinv_col, mask_col, y_pad)
    return _solve_predict(g_mat, r_mat, xg)


# speedup vs baseline: 1.4479x; 1.1886x over previous
import functools

import jax
import jax.numpy as jnp
from jax import lax
from jax.experimental import pallas as pl
from jax.experimental.pallas import tpu as pltpu
from jax.experimental.pallas import tpu_sc as plsc

N = 10000
E = 320000
D = 128
C = 16
ALPHA = 0.1
REG = 1e-05

NW = 32
NP = 320
N_PAD = NW * NP
CH = 128
NPT = N_PAD // 16
TRASH = 128
E_PAD = 327680
EB_A = 20
NB0 = 11
NB1 = 9


def _rsqrt16(v):
    i = lax.bitcast_convert_type(v, jnp.int32)
    i = jnp.int32(0x5F3759DF) - (i >> 1)
    y = lax.bitcast_convert_type(i, jnp.float32)
    for _ in range(3):
        y = y * (1.5 - 0.5 * v * y * y)
    return y


def _deg_dinv_s0_body(col3d_hbm, x_hbm, dinv_hbm, s0_hbm,
                      deg_sh, zbuf, ones_v, ci_all, degv, xv, sem_i, sem_s):
    cid = lax.axis_index("c")
    sid = lax.axis_index("s")
    wid = cid * 16 + sid

    for g in range(NPT // 16):
        zbuf[pl.ds(16 * g, 16)] = jnp.zeros((16,), jnp.float32)
    pltpu.sync_copy(zbuf, deg_sh.at[pl.ds(sid * NPT, NPT)])
    pltpu.sync_copy(zbuf.at[pl.ds(0, 8)],
                    deg_sh.at[pl.ds(N_PAD + sid * 8, 8)])
    for g in range(CH // 16):
        ones_v[pl.ds(16 * g, 16)] = jnp.full((16,), 1.0, jnp.float32)
    plsc.subcore_barrier()

    pltpu.async_copy(col3d_hbm.at[pl.ds(sid * EB_A, EB_A)], ci_all,
                     sem_i).wait()
    descs = []
    for b in range(EB_A):
        for t in range(8):
            descs.append(pltpu.async_copy(
                ones_v, deg_sh.at[ci_all.at[b, t]], sem_s, add=True))
    for dsc in descs:
        dsc.wait()
    plsc.subcore_barrier()

    base = jnp.minimum(wid * NP, N - NP)
    pltpu.sync_copy(deg_sh.at[pl.ds(base, NP)], degv)
    for g in range(NP // 16):
        dv = degv[pl.ds(16 * g, 16)] + 1.0
        degv[pl.ds(16 * g, 16)] = _rsqrt16(dv)
    pltpu.sync_copy(degv, dinv_hbm.at[pl.ds(base, NP)])

    pltpu.sync_copy(x_hbm.at[pl.ds(base, NP)], xv)

    def scale_group(g, _):
        dvec = degv[pl.ds(16 * g, 16)]
        for l in range(16):
            dv = dvec[l]
            r = 16 * g + l
            for j in range(D // 16):
                xv[r, pl.ds(16 * j, 16)] = xv[r, pl.ds(16 * j, 16)] * dv
        return _

    lax.fori_loop(0, NP // 16, scale_group, None)
    pltpu.sync_copy(xv, s0_hbm.at[pl.ds(base, NP)])


def _deg_dinv_s0(col3d, x_pad):
    mesh = plsc.VectorSubcoreMesh(core_axis_name="c", subcore_axis_name="s")
    return pl.kernel(
        _deg_dinv_s0_body,
        out_type=(
            jax.ShapeDtypeStruct((N,), jnp.float32),
            jax.ShapeDtypeStruct((N, D), jnp.float32),
        ),
        mesh=mesh,
        scratch_types=[
            pltpu.VMEM_SHARED((N_PAD + TRASH,), jnp.float32),
            pltpu.VMEM((NPT,), jnp.float32),
            pltpu.VMEM((CH,), jnp.float32),
            pltpu.VMEM((EB_A, 8, CH), jnp.int32),
            pltpu.VMEM((NP,), jnp.float32),
            pltpu.VMEM((NP, D), jnp.float32),
            pltpu.SemaphoreType.DMA,
            pltpu.SemaphoreType.DMA,
        ],
    )(col3d, x_pad)


def _hop_body(s_hbm, row3d_hbm, col3d_hbm, tp_hbm,
              acc_sh, ri0, ci0, ri1, ci1, rows0, rows1,
              sem_i0, sem_i1, sem_g0, sem_g1):
    cid = lax.axis_index("c")
    sid = lax.axis_index("s")

    def zrow(r, _):
        for g in range(D // 16):
            rows0[r, pl.ds(16 * g, 16)] = jnp.zeros((16,), jnp.float32)
        return _

    lax.fori_loop(0, CH, zrow, None)
    for b in range(NPT // CH):
        pltpu.sync_copy(rows0, acc_sh.at[pl.ds(sid * NPT + b * CH, CH)])
    pltpu.sync_copy(rows0.at[pl.ds(0, 8)],
                    acc_sh.at[pl.ds(N_PAD + sid * 8, 8)])

    ibufs = ((ri0, ci0), (ri1, ci1))
    isems = (sem_i0, sem_i1)
    gbufs = (rows0, rows1)
    gsems = (sem_g0, sem_g1)

    def run_pipe(tb, nblk):
        def fire_idx(blk, par):
            dr = pltpu.async_copy(row3d_hbm.at[tb + blk], ibufs[par][0],
                                  isems[par])
            dc = pltpu.async_copy(col3d_hbm.at[tb + blk], ibufs[par][1],
                                  isems[par])
            return (dr, dc)

        pend_i = [fire_idx(0, 0), None]
        for dsc in pend_i[0]:
            dsc.wait()
        if nblk > 1:
            pend_i[1] = fire_idx(1, 1)
        n_ch = nblk * 8
        pend_g = pltpu.async_copy(s_hbm.at[ri0.at[0]], rows0, sem_g0)
        for k in range(n_ch):
            b, t = divmod(k, 8)
            par = b % 2
            cur_buf = gbufs[k % 2]
            cur_dsc = pend_g
            if k + 1 < n_ch:
                nb, nt = divmod(k + 1, 8)
                if nt == 0:
                    for dsc in pend_i[nb % 2]:
                        dsc.wait()
                pend_g = pltpu.async_copy(s_hbm.at[ibufs[nb % 2][0].at[nt]],
                                          gbufs[(k + 1) % 2],
                                          gsems[(k + 1) % 2])
            cur_dsc.wait()
            pltpu.sync_copy(cur_buf, acc_sh.at[ibufs[par][1].at[t]], add=True)
            if t == 7 and b + 2 < nblk:
                pend_i[par] = fire_idx(b + 2, par)

    @pl.when(cid == 0)
    def _():
        run_pipe(sid * NB0, NB0)

    @pl.when(cid == 1)
    def _():
        run_pipe(16 * NB0 + sid * NB1, NB1)

    plsc.subcore_barrier()

    pltpu.sync_copy(acc_sh.at[pl.ds(sid * NPT, NPT)],
                    tp_hbm.at[cid, pl.ds(sid * NPT, NPT)])


def _hop(s, row3d, col3d):
    mesh = plsc.VectorSubcoreMesh(core_axis_name="c", subcore_axis_name="s")
    return pl.kernel(
        _hop_body,
        out_type=jax.ShapeDtypeStruct((2, N_PAD, D), jnp.float32),
        mesh=mesh,
        scratch_types=[
            pltpu.VMEM_SHARED((N_PAD + TRASH, D), jnp.float32),
            pltpu.VMEM((8, CH), jnp.int32),
            pltpu.VMEM((8, CH), jnp.int32),
            pltpu.VMEM((8, CH), jnp.int32),
            pltpu.VMEM((8, CH), jnp.int32),
            pltpu.VMEM((CH, D), jnp.float32),
            pltpu.VMEM((CH, D), jnp.float32),
            pltpu.SemaphoreType.DMA,
            pltpu.SemaphoreType.DMA,
            pltpu.SemaphoreType.DMA,
            pltpu.SemaphoreType.DMA,
        ],
    )(s, row3d, col3d)


BN = 2000
NBLK = N // BN


def _combine1_body(tp, s, x, dinv, s_next):
    t = tp[0] + tp[1] + s[...]
    dv = dinv[...]
    out = (1.0 - ALPHA) * (dv * t) + ALPHA * x[...]
    s_next[...] = dv * out


def _combine1(tp, s, x, dinv_col):
    return pl.pallas_call(
        _combine1_body,
        grid=(NBLK,),
        in_specs=[
            pl.BlockSpec((2, BN, D), lambda i: (0, i, 0)),
            pl.BlockSpec((BN, D), lambda i: (i, 0)),
            pl.BlockSpec((BN, D), lambda i: (i, 0)),
            pl.BlockSpec((BN, 1), lambda i: (i, 0)),
        ],
        out_specs=pl.BlockSpec((BN, D), lambda i: (i, 0)),
        out_shape=jax.ShapeDtypeStruct((N, D), jnp.float32),
    )(tp, s, x, dinv_col)


def _combine_solve_body(tp, s, x, dinv, mask, yb, yp, xg_sc, acc_g, acc_r):
    i = pl.program_id(0)

    @pl.when(i == 0)
    def _():
        acc_g[...] = jnp.zeros_like(acc_g)
        acc_r[...] = jnp.zeros_like(acc_r)

    @pl.when(i < NBLK)
    def _():
        t = tp[0] + tp[1] + s[...]
        xg = (1.0 - ALPHA) * (dinv[...] * t) + ALPHA * x[...]
        xg_sc[pl.ds(i * BN, BN)] = xg
        xm = xg * mask[...]
        acc_g[...] += lax.dot_general(xm, xg, (((0,), (0,)), ((), ())),
                                      preferred_element_type=jnp.float32)
        acc_r[...] += lax.dot_general(xm, yb[...], (((0,), (0,)), ((), ())),
                                      preferred_element_type=jnp.float32)

    @pl.when(i == NBLK)
    def _():
        rows = lax.broadcasted_iota(jnp.int32, (D, D), 0)
        cols = lax.broadcasted_iota(jnp.int32, (D, D), 1)
        a = acc_g[...] + jnp.where(rows == cols, jnp.float32(REG), 0.0)
        aabs = jnp.abs(a)
        n1 = jnp.max(jnp.sum(aabs, axis=0))
        ninf = jnp.max(jnp.sum(aabs, axis=1))
        two_i = jnp.where(rows == cols, jnp.float32(2.0), jnp.float32(0.0))
        xinv = a * (1.0 / (n1 * ninf))
        for _ in range(24):
            ax = lax.dot_general(a, xinv, (((1,), (0,)), ((), ())),
                                 preferred_element_type=jnp.float32,
                                 precision=lax.Precision.HIGHEST)
            xinv = lax.dot_general(xinv, two_i - ax, (((1,), (0,)), ((), ())),
                                   preferred_element_type=jnp.float32,
                                   precision=lax.Precision.HIGHEST)
        sol = lax.dot_general(xinv, acc_r[...], (((1,), (0,)), ((), ())),
                              preferred_element_type=jnp.float32)
        yp[...] = lax.dot_general(xg_sc[...], sol, (((1,), (0,)), ((), ())),
                                  preferred_element_type=jnp.float32)


def _combine_solve(tp, s, x, dinv_col, mask_col, y_one_hot):
    clamp = lambda i: jnp.minimum(i, NBLK - 1)
    return pl.pallas_call(
        _combine_solve_body,
        grid=(NBLK + 1,),
        in_specs=[
            pl.BlockSpec((2, BN, D), lambda i: (0, clamp(i), 0)),
            pl.BlockSpec((BN, D), lambda i: (clamp(i), 0)),
            pl.BlockSpec((BN, D), lambda i: (clamp(i), 0)),
            pl.BlockSpec((BN, 1), lambda i: (clamp(i), 0)),
            pl.BlockSpec((BN, 1), lambda i: (clamp(i), 0)),
            pl.BlockSpec((BN, C), lambda i: (clamp(i), 0)),
        ],
        out_specs=pl.BlockSpec((N, C), lambda i: (0, 0)),
        out_shape=jax.ShapeDtypeStruct((N, C), jnp.float32),
        scratch_shapes=[
            pltpu.VMEM((N, D), jnp.float32),
            pltpu.VMEM((D, D), jnp.float32),
            pltpu.VMEM((D, C), jnp.float32),
        ],
    )(tp, s, x, dinv_col, mask_col, y_one_hot)


def kernel(x, edge_index, y_one_hot, train_mask):
    i = jnp.arange(E_PAD - E, dtype=jnp.int32)
    row3d = jnp.concatenate([edge_index[0], (i * 41) % N]).reshape(-1, 8, CH)
    col3d = jnp.concatenate([edge_index[1],
                             N_PAD + (i % TRASH)]).reshape(-1, 8, CH)
    mask_col = train_mask.astype(jnp.float32)[:, None]

    dinv, s0 = _deg_dinv_s0(col3d, x)
    dinv_col = dinv[:, None]

    tp1 = _hop(s0, row3d, col3d)
    s1 = _combine1(tp1, s0, x, dinv_col)
    tp2 = _hop(s1, row3d, col3d)
    return _combine_solve(tp2, s1, x, dinv_col, mask_col, y_one_hot)
